# baseline (speedup 1.0000x reference)
import functools

import numpy as np
import jax
from jax import lax
import jax.numpy as jnp
from jax.experimental import pallas as pl
from jax.experimental.pallas import tpu as pltpu
from jax.experimental.pallas import tpu_sc as plsc

_N_ATOMS = 4096
_N_MOLS = 128
_N_ELEM = 4
_NSTACKS = 32
_NPCAS = 128
_SIGMA = 3.0
_NFEAT = _NSTACKS * _NPCAS

_B = 256
_NBLK = _N_ATOMS // _B

_COEFF_NORM = np.float32(np.sqrt(np.float32(_NPCAS)) / _SIGMA)


def _hadamard(n):
    H = np.array([[1.0]], dtype=np.float64)
    while H.shape[0] < n:
        H = np.block([[H, H], [H, -H]])
    return H


_R = _B * _NSTACKS


def _tile(t):
    return jnp.broadcast_to(t[None], (_B, _NSTACKS, _NPCAS)).reshape(
        _R, _NPCAS)


def _tc_body(rep_ref, d0_ref, d1_ref, bias_ref, alpha_ref, hn_ref,
             zs_ref, e_ref):
    zs = zs_ref[0]
    zoh = (zs == lax.broadcasted_iota(jnp.int32, (1, _NPCAS), 1)
           ).astype(jnp.bfloat16)
    d0 = lax.dot(zoh, d0_ref[...],
                 preferred_element_type=jnp.float32
                 ).astype(jnp.bfloat16)
    d1 = lax.dot(zoh, d1_ref[...], preferred_element_type=jnp.float32)

    z_col = lax.shift_right_logical(zs, 5)
    r01 = jnp.where(z_col == 0, _tile(bias_ref[0]), _tile(bias_ref[1]))
    r23 = jnp.where(z_col == 2, _tile(bias_ref[2]), _tile(bias_ref[3]))
    b = jnp.where(z_col <= 1, r01, r23)

    rep = rep_ref[...].astype(jnp.bfloat16)
    rep_x = jnp.broadcast_to(rep[:, None, :],
                             (_B, _NSTACKS, _NPCAS)).reshape(_R, _NPCAS)

    hn = hn_ref[...]
    v = lax.dot(rep_x * d0, hn, preferred_element_type=jnp.float32)
    v = (v * d1).astype(jnp.bfloat16)
    v = lax.dot(v, hn, preferred_element_type=jnp.float32)

    arg = _COEFF_NORM * v + b
    w = jnp.cos(arg)
    w = w * _tile(alpha_ref[...])
    e = jnp.sum(w.reshape(_B, _NSTACKS, _NPCAS), axis=(1, 2))
    e_ref[...] = e.reshape(1, 1, _B)


_NSUB = 16
_CHUNK = _N_ATOMS // _NSUB
_L = 16


def _sc_segsum(e_hbm, mol_hbm, out_hbm, e_v, mol_v, acc2_v, part_v, stage_v,
               shared):
    c = lax.axis_index("c")
    s = lax.axis_index("s")

    @pl.when(c == 0)
    def _():
        base = s * _CHUNK
        pltpu.sync_copy(e_hbm.at[pl.ds(base, _CHUNK)], e_v)
        pltpu.sync_copy(mol_hbm.at[pl.ds(base, _CHUNK)], mol_v)

        zero16 = jnp.zeros((_L,), jnp.float32)
        for j in range(_L * _N_MOLS // _L):
            acc2_v[pl.ds(j * _L, _L)] = zero16

        rowoff = lax.iota(jnp.int32, _L) * _N_MOLS

        def body(i, carry):
            ids = mol_v[pl.ds(i * _L, _L)]
            vals = e_v[pl.ds(i * _L, _L)]
            plsc.addupdate_scatter(acc2_v, [ids + rowoff], vals)
            return carry

        lax.fori_loop(0, _CHUNK // _L, body, 0)

        for k in range(_N_MOLS // _L):
            ssum = zero16
            for r in range(_L):
                ssum = ssum + acc2_v[pl.ds(r * _N_MOLS + k * _L, _L)]
            part_v[pl.ds(k * _L, _L)] = ssum

        pltpu.sync_copy(part_v, shared.at[s])
        plsc.subcore_barrier()

        @pl.when(s == 0)
        def _():
            pltpu.sync_copy(shared, stage_v)
            for k in range(_N_MOLS // _L):
                ssum2 = jnp.zeros((_L,), jnp.float32)
                for r in range(_NSUB):
                    ssum2 = ssum2 + stage_v[r, pl.ds(k * _L, _L)]
                part_v[pl.ds(k * _L, _L)] = ssum2
            pltpu.sync_copy(part_v, out_hbm)


_SC_SEGSUM_CACHE = []


def _get_sc_segsum():
    if not _SC_SEGSUM_CACHE:
        k = functools.partial(
            pl.kernel,
            mesh=plsc.VectorSubcoreMesh(core_axis_name="c",
                                        subcore_axis_name="s"),
            out_type=jax.ShapeDtypeStruct((_N_MOLS,), jnp.float32),
            scratch_types=[
                pltpu.VMEM((_CHUNK,), jnp.float32),
                pltpu.VMEM((_CHUNK,), jnp.int32),
                pltpu.VMEM((_L * _N_MOLS,), jnp.float32),
                pltpu.VMEM((_N_MOLS,), jnp.float32),
                pltpu.VMEM((_NSUB, _N_MOLS), jnp.float32),
                pltpu.VMEM_SHARED((_NSUB, _N_MOLS), jnp.float32),
            ],
            compiler_params=pltpu.CompilerParams(needs_layout_passes=False),
        )(_sc_segsum)
        _SC_SEGSUM_CACHE.append(k)
    return _SC_SEGSUM_CACHE[0]


def kernel(rep, Dmat, bias, alpha, Z, mol_ids):
    hn = jnp.asarray(_hadamard(_NPCAS) / np.sqrt(_NPCAS),
                     dtype=jnp.float32).astype(jnp.bfloat16)
    alpha_s = (alpha * np.float32(np.sqrt(2.0 / _NFEAT))).reshape(
        _NSTACKS, _NPCAS)

    d0 = Dmat[:, 0].reshape(_N_ELEM * _NSTACKS, _NPCAS).astype(jnp.bfloat16)
    d1 = Dmat[:, 1].reshape(_N_ELEM * _NSTACKS, _NPCAS).astype(jnp.bfloat16)
    bias_t = bias.reshape(_N_ELEM, _NSTACKS, _NPCAS)
    zs3 = (Z[:, None] * _NSTACKS + jnp.arange(_NSTACKS, dtype=jnp.int32)
           ).reshape(_NBLK, _R, 1)

    e = pl.pallas_call(
        _tc_body,
        grid=(_NBLK,),
        in_specs=[
            pl.BlockSpec((_B, _NPCAS), lambda i: (i, 0)),
            pl.BlockSpec((_N_ELEM * _NSTACKS, _NPCAS), lambda i: (0, 0)),
            pl.BlockSpec((_N_ELEM * _NSTACKS, _NPCAS), lambda i: (0, 0)),
            pl.BlockSpec((_N_ELEM, _NSTACKS, _NPCAS), lambda i: (0, 0, 0)),
            pl.BlockSpec((_NSTACKS, _NPCAS), lambda i: (0, 0)),
            pl.BlockSpec((_NPCAS, _NPCAS), lambda i: (0, 0)),
            pl.BlockSpec((1, _R, 1), lambda i: (i, 0, 0)),
        ],
        out_specs=pl.BlockSpec((1, 1, _B), lambda i: (i, 0, 0)),
        out_shape=jax.ShapeDtypeStruct((_NBLK, 1, _B), jnp.float32),
        compiler_params=pltpu.CompilerParams(
            dimension_semantics=("arbitrary",),
        ),
    )(rep, d0, d1, bias_t, alpha_s, hn, zs3)

    return _get_sc_segsum()(e.reshape(_N_ATOMS), mol_ids)


# speedup vs baseline: 3.8491x; 3.8491x over previous
import functools

import numpy as np
import jax
from jax import lax
import jax.numpy as jnp
from jax.experimental import pallas as pl
from jax.experimental.pallas import tpu as pltpu
from jax.experimental.pallas import tpu_sc as plsc

_N_ATOMS = 4096
_N_MOLS = 128
_N_ELEM = 4
_NSTACKS = 32
_NPCAS = 128
_SIGMA = 3.0
_NFEAT = _NSTACKS * _NPCAS

_B = 256
_NBLK = _N_ATOMS // _B

_COEFF_NORM = np.float32(np.sqrt(np.float32(_NPCAS)) / _SIGMA)


def _hadamard(n):
    H = np.array([[1.0]], dtype=np.float64)
    while H.shape[0] < n:
        H = np.block([[H, H], [H, -H]])
    return H


_R = _B * _NSTACKS


def _tile(t):
    return jnp.broadcast_to(t[None], (_B, _NSTACKS, _NPCAS)).reshape(
        _R, _NPCAS)


_COS_COEF = (6.5281506, -25.964163, 60.16561, -85.4497, 64.93908,
             -19.739202, 1.0)
_MAGIC = np.float32(1.5 * 2 ** 23)
_K = np.float32(_COEFF_NORM / (2.0 * np.pi))


def _select4(z_col, tbl_ref):
    r01 = jnp.where(z_col == 0, _tile(tbl_ref[0]), _tile(tbl_ref[1]))
    r23 = jnp.where(z_col == 2, _tile(tbl_ref[2]), _tile(tbl_ref[3]))
    return jnp.where(z_col <= 1, r01, r23)


def _tc_body(rep_ref, d0_ref, d1_ref, bias2_ref, alpha_ref, hn_ref,
             z_ref, e_ref):
    z = z_ref[0, 0, :].reshape(_B, 1)
    z_exp = jnp.broadcast_to(z.reshape(_B, 1, 1),
                             (_B, _NSTACKS, 1)).reshape(_R, 1)

    d0 = _select4(z_exp, d0_ref)
    d1 = _select4(z_exp, d1_ref)
    b2 = _select4(z_exp, bias2_ref)

    rep = rep_ref[...].astype(jnp.bfloat16)
    rep_x = jnp.broadcast_to(rep[:, None, :],
                             (_B, _NSTACKS, _NPCAS)).reshape(_R, _NPCAS)

    hn = hn_ref[...]
    v = lax.dot(rep_x * d0, hn, preferred_element_type=jnp.float32)
    v = (v * d1).astype(jnp.bfloat16)
    v = lax.dot(v, hn, preferred_element_type=jnp.float32)

    t = v * _K + b2
    r = t - jnp.round(t)
    u = r * r
    w = jnp.float32(_COS_COEF[0])
    for c in _COS_COEF[1:]:
        w = w * u + jnp.float32(c)
    w = w * _tile(alpha_ref[...])
    e = jnp.sum(w.reshape(_B, _NSTACKS, _NPCAS), axis=(1, 2))
    e_ref[...] = e.reshape(1, 1, _B)


_NSUB = 16
_CHUNK = _N_ATOMS // _NSUB
_L = 16


def _sc_segsum(e_hbm, mol_hbm, out_hbm, e_v, mol_v, acc2_v, part_v, stage_v,
               shared):
    c = lax.axis_index("c")
    s = lax.axis_index("s")

    @pl.when(c == 0)
    def _():
        base = s * _CHUNK
        pltpu.sync_copy(e_hbm.at[pl.ds(base, _CHUNK)], e_v)
        pltpu.sync_copy(mol_hbm.at[pl.ds(base, _CHUNK)], mol_v)

        zero16 = jnp.zeros((_L,), jnp.float32)
        for j in range(_L * _N_MOLS // _L):
            acc2_v[pl.ds(j * _L, _L)] = zero16

        rowoff = lax.iota(jnp.int32, _L) * _N_MOLS

        def body(i, carry):
            ids = mol_v[pl.ds(i * _L, _L)]
            vals = e_v[pl.ds(i * _L, _L)]
            plsc.addupdate_scatter(acc2_v, [ids + rowoff], vals)
            return carry

        lax.fori_loop(0, _CHUNK // _L, body, 0)

        for k in range(_N_MOLS // _L):
            ssum = zero16
            for r in range(_L):
                ssum = ssum + acc2_v[pl.ds(r * _N_MOLS + k * _L, _L)]
            part_v[pl.ds(k * _L, _L)] = ssum

        pltpu.sync_copy(part_v, shared.at[s])
        plsc.subcore_barrier()

        @pl.when(s == 0)
        def _():
            pltpu.sync_copy(shared, stage_v)
            for k in range(_N_MOLS // _L):
                ssum2 = jnp.zeros((_L,), jnp.float32)
                for r in range(_NSUB):
                    ssum2 = ssum2 + stage_v[r, pl.ds(k * _L, _L)]
                part_v[pl.ds(k * _L, _L)] = ssum2
            pltpu.sync_copy(part_v, out_hbm)


_SC_SEGSUM_CACHE = []


def _get_sc_segsum():
    if not _SC_SEGSUM_CACHE:
        k = functools.partial(
            pl.kernel,
            mesh=plsc.VectorSubcoreMesh(core_axis_name="c",
                                        subcore_axis_name="s"),
            out_type=jax.ShapeDtypeStruct((_N_MOLS,), jnp.float32),
            scratch_types=[
                pltpu.VMEM((_CHUNK,), jnp.float32),
                pltpu.VMEM((_CHUNK,), jnp.int32),
                pltpu.VMEM((_L * _N_MOLS,), jnp.float32),
                pltpu.VMEM((_N_MOLS,), jnp.float32),
                pltpu.VMEM((_NSUB, _N_MOLS), jnp.float32),
                pltpu.VMEM_SHARED((_NSUB, _N_MOLS), jnp.float32),
            ],
            compiler_params=pltpu.CompilerParams(needs_layout_passes=False),
        )(_sc_segsum)
        _SC_SEGSUM_CACHE.append(k)
    return _SC_SEGSUM_CACHE[0]


def kernel(rep, Dmat, bias, alpha, Z, mol_ids):
    hn = jnp.asarray(_hadamard(_NPCAS) / np.sqrt(_NPCAS),
                     dtype=jnp.float32).astype(jnp.bfloat16)
    alpha_s = (alpha * np.float32(np.sqrt(2.0 / _NFEAT))).reshape(
        _NSTACKS, _NPCAS)

    d0 = Dmat[:, 0].astype(jnp.bfloat16)
    d1 = Dmat[:, 1].astype(jnp.bfloat16)
    bias2_t = (bias * np.float32(1.0 / (2.0 * np.pi))).reshape(
        _N_ELEM, _NSTACKS, _NPCAS)
    z3 = Z.reshape(_NBLK, 1, _B)

    e = pl.pallas_call(
        _tc_body,
        grid=(_NBLK,),
        in_specs=[
            pl.BlockSpec((_B, _NPCAS), lambda i: (i, 0)),
            pl.BlockSpec((_N_ELEM, _NSTACKS, _NPCAS), lambda i: (0, 0, 0)),
            pl.BlockSpec((_N_ELEM, _NSTACKS, _NPCAS), lambda i: (0, 0, 0)),
            pl.BlockSpec((_N_ELEM, _NSTACKS, _NPCAS), lambda i: (0, 0, 0)),
            pl.BlockSpec((_NSTACKS, _NPCAS), lambda i: (0, 0)),
            pl.BlockSpec((_NPCAS, _NPCAS), lambda i: (0, 0)),
            pl.BlockSpec((1, 1, _B), lambda i: (i, 0, 0)),
        ],
        out_specs=pl.BlockSpec((1, 1, _B), lambda i: (i, 0, 0)),
        out_shape=jax.ShapeDtypeStruct((_NBLK, 1, _B), jnp.float32),
        compiler_params=pltpu.CompilerParams(
            dimension_semantics=("arbitrary",),
        ),
    )(rep, d0, d1, bias2_t, alpha_s, hn, z3)

    return _get_sc_segsum()(e.reshape(_N_ATOMS), mol_ids)


# speedup vs baseline: 3.8894x; 1.0105x over previous
import functools

import numpy as np
import jax
from jax import lax
import jax.numpy as jnp
from jax.experimental import pallas as pl
from jax.experimental.pallas import tpu as pltpu
from jax.experimental.pallas import tpu_sc as plsc

_N_ATOMS = 4096
_N_MOLS = 128
_N_ELEM = 4
_NSTACKS = 32
_NPCAS = 128
_SIGMA = 3.0
_NFEAT = _NSTACKS * _NPCAS

_B = 256
_NBLK = _N_ATOMS // _B

_COEFF_NORM = np.float32(np.sqrt(np.float32(_NPCAS)) / _SIGMA)


def _hadamard(n):
    H = np.array([[1.0]], dtype=np.float64)
    while H.shape[0] < n:
        H = np.block([[H, H], [H, -H]])
    return H


_R = _B * _NSTACKS


def _tile(t):
    return jnp.broadcast_to(t[None], (_B, _NSTACKS, _NPCAS)).reshape(
        _R, _NPCAS)


_COS_COEF = (-21.06805, 58.774673, -85.26424, 64.92748, -19.738913,
             0.9999983)
_K = np.float32(_COEFF_NORM / (2.0 * np.pi))


def _select4(z_col, tbl_ref):
    r01 = jnp.where(z_col == 0, _tile(tbl_ref[0]), _tile(tbl_ref[1]))
    r23 = jnp.where(z_col == 2, _tile(tbl_ref[2]), _tile(tbl_ref[3]))
    return jnp.where(z_col <= 1, r01, r23)


def _tc_body(rep_ref, d0_ref, d1_ref, bias2_ref, alpha_ref, hn_ref,
             z_ref, e_ref):
    z = z_ref[0, 0, :].reshape(_B, 1)
    z_exp = jnp.broadcast_to(z.reshape(_B, 1, 1),
                             (_B, _NSTACKS, 1)).reshape(_R, 1)

    d0 = _select4(z_exp, d0_ref)
    d1 = _select4(z_exp, d1_ref)
    b2 = _select4(z_exp, bias2_ref)

    rep = rep_ref[...].astype(jnp.bfloat16)
    rep_x = jnp.broadcast_to(rep[:, None, :],
                             (_B, _NSTACKS, _NPCAS)).reshape(_R, _NPCAS)

    hn = hn_ref[...]
    v = lax.dot(rep_x * d0, hn, preferred_element_type=jnp.float32)
    v = v.astype(jnp.bfloat16) * d1
    v = lax.dot(v, hn, preferred_element_type=jnp.float32)

    t = v * _K + b2
    r = t - jnp.round(t)
    u = r * r
    w = jnp.float32(_COS_COEF[0])
    for c in _COS_COEF[1:]:
        w = w * u + jnp.float32(c)
    w = w * _tile(alpha_ref[...])
    ws = jnp.sum(w.reshape(_B, _NSTACKS, _NPCAS), axis=1)
    e = lax.dot(ws, jnp.ones((_NPCAS, 1), jnp.float32),
                precision=jax.lax.Precision.HIGHEST)
    e_ref[...] = e.reshape(1, 1, _B)


_NSUB = 16
_CHUNK = _N_ATOMS // _NSUB
_L = 16


def _sc_segsum(e_hbm, mol_hbm, out_hbm, e_v, mol_v, acc2_v, part_v, stage_v,
               shared):
    c = lax.axis_index("c")
    s = lax.axis_index("s")

    @pl.when(c == 0)
    def _():
        base = s * _CHUNK
        pltpu.sync_copy(e_hbm.at[pl.ds(base, _CHUNK)], e_v)
        pltpu.sync_copy(mol_hbm.at[pl.ds(base, _CHUNK)], mol_v)

        zero16 = jnp.zeros((_L,), jnp.float32)
        for j in range(_L * _N_MOLS // _L):
            acc2_v[pl.ds(j * _L, _L)] = zero16

        rowoff = lax.iota(jnp.int32, _L) * _N_MOLS

        def body(i, carry):
            ids = mol_v[pl.ds(i * _L, _L)]
            vals = e_v[pl.ds(i * _L, _L)]
            plsc.addupdate_scatter(acc2_v, [ids + rowoff], vals)
            return carry

        lax.fori_loop(0, _CHUNK // _L, body, 0)

        for k in range(_N_MOLS // _L):
            ssum = zero16
            for r in range(_L):
                ssum = ssum + acc2_v[pl.ds(r * _N_MOLS + k * _L, _L)]
            part_v[pl.ds(k * _L, _L)] = ssum

        pltpu.sync_copy(part_v, shared.at[s])
        plsc.subcore_barrier()

        @pl.when(s == 0)
        def _():
            pltpu.sync_copy(shared, stage_v)
            for k in range(_N_MOLS // _L):
                ssum2 = jnp.zeros((_L,), jnp.float32)
                for r in range(_NSUB):
                    ssum2 = ssum2 + stage_v[r, pl.ds(k * _L, _L)]
                part_v[pl.ds(k * _L, _L)] = ssum2
            pltpu.sync_copy(part_v, out_hbm)


_SC_SEGSUM_CACHE = []


def _get_sc_segsum():
    if not _SC_SEGSUM_CACHE:
        k = functools.partial(
            pl.kernel,
            mesh=plsc.VectorSubcoreMesh(core_axis_name="c",
                                        subcore_axis_name="s"),
            out_type=jax.ShapeDtypeStruct((_N_MOLS,), jnp.float32),
            scratch_types=[
                pltpu.VMEM((_CHUNK,), jnp.float32),
                pltpu.VMEM((_CHUNK,), jnp.int32),
                pltpu.VMEM((_L * _N_MOLS,), jnp.float32),
                pltpu.VMEM((_N_MOLS,), jnp.float32),
                pltpu.VMEM((_NSUB, _N_MOLS), jnp.float32),
                pltpu.VMEM_SHARED((_NSUB, _N_MOLS), jnp.float32),
            ],
            compiler_params=pltpu.CompilerParams(needs_layout_passes=False),
        )(_sc_segsum)
        _SC_SEGSUM_CACHE.append(k)
    return _SC_SEGSUM_CACHE[0]


def kernel(rep, Dmat, bias, alpha, Z, mol_ids):
    hn = jnp.asarray(_hadamard(_NPCAS) / np.sqrt(_NPCAS),
                     dtype=jnp.float32).astype(jnp.bfloat16)
    alpha_s = (alpha * np.float32(np.sqrt(2.0 / _NFEAT))).reshape(
        _NSTACKS, _NPCAS)

    d0 = Dmat[:, 0].astype(jnp.bfloat16)
    d1 = Dmat[:, 1].astype(jnp.bfloat16)
    bias2_t = (bias * np.float32(1.0 / (2.0 * np.pi))).reshape(
        _N_ELEM, _NSTACKS, _NPCAS)
    z3 = Z.reshape(_NBLK, 1, _B)

    e = pl.pallas_call(
        _tc_body,
        grid=(_NBLK,),
        in_specs=[
            pl.BlockSpec((_B, _NPCAS), lambda i: (i, 0)),
            pl.BlockSpec((_N_ELEM, _NSTACKS, _NPCAS), lambda i: (0, 0, 0)),
            pl.BlockSpec((_N_ELEM, _NSTACKS, _NPCAS), lambda i: (0, 0, 0)),
            pl.BlockSpec((_N_ELEM, _NSTACKS, _NPCAS), lambda i: (0, 0, 0)),
            pl.BlockSpec((_NSTACKS, _NPCAS), lambda i: (0, 0)),
            pl.BlockSpec((_NPCAS, _NPCAS), lambda i: (0, 0)),
            pl.BlockSpec((1, 1, _B), lambda i: (i, 0, 0)),
        ],
        out_specs=pl.BlockSpec((1, 1, _B), lambda i: (i, 0, 0)),
        out_shape=jax.ShapeDtypeStruct((_NBLK, 1, _B), jnp.float32),
        compiler_params=pltpu.CompilerParams(
            dimension_semantics=("arbitrary",),
        ),
    )(rep, d0, d1, bias2_t, alpha_s, hn, z3)

    return _get_sc_segsum()(e.reshape(_N_ATOMS), mol_ids)


# speedup vs baseline: 3.9430x; 1.0138x over previous
import functools

import numpy as np
import jax
from jax import lax
import jax.numpy as jnp
from jax.experimental import pallas as pl
from jax.experimental.pallas import tpu as pltpu
from jax.experimental.pallas import tpu_sc as plsc

_N_ATOMS = 4096
_N_MOLS = 128
_N_ELEM = 4
_NSTACKS = 32
_NPCAS = 128
_SIGMA = 3.0
_NFEAT = _NSTACKS * _NPCAS

_B = 512
_NBLK = _N_ATOMS // _B

_COEFF_NORM = np.float32(np.sqrt(np.float32(_NPCAS)) / _SIGMA)


def _hadamard(n):
    H = np.array([[1.0]], dtype=np.float64)
    while H.shape[0] < n:
        H = np.block([[H, H], [H, -H]])
    return H


_R = _B * _NSTACKS


def _tile(t):
    return jnp.broadcast_to(t[None], (_B, _NSTACKS, _NPCAS)).reshape(
        _R, _NPCAS)


_COS_COEF = (-21.06805, 58.774673, -85.26424, 64.92748, -19.738913,
             0.9999983)
_K = np.float32(_COEFF_NORM / (2.0 * np.pi))


def _select4(z_col, tbl_ref):
    r01 = jnp.where(z_col == 0, _tile(tbl_ref[0]), _tile(tbl_ref[1]))
    r23 = jnp.where(z_col == 2, _tile(tbl_ref[2]), _tile(tbl_ref[3]))
    return jnp.where(z_col <= 1, r01, r23)


def _tc_body(rep_ref, d0_ref, d1_ref, bias2_ref, alpha_ref, hn_ref,
             z_ref, e_ref):
    z = z_ref[0, 0, :].reshape(_B, 1)
    z_exp = jnp.broadcast_to(z.reshape(_B, 1, 1),
                             (_B, _NSTACKS, 1)).reshape(_R, 1)

    d0 = _select4(z_exp, d0_ref)
    d1 = _select4(z_exp, d1_ref)
    b2 = _select4(z_exp, bias2_ref)

    rep = rep_ref[...].astype(jnp.bfloat16)
    rep_x = jnp.broadcast_to(rep[:, None, :],
                             (_B, _NSTACKS, _NPCAS)).reshape(_R, _NPCAS)

    hn = hn_ref[...]
    v = lax.dot(rep_x * d0, hn, preferred_element_type=jnp.float32)
    v = v.astype(jnp.bfloat16) * d1
    v = lax.dot(v, hn, preferred_element_type=jnp.float32)

    t = v * _K + b2
    r = t - jnp.round(t)
    u = r * r
    w = jnp.float32(_COS_COEF[0])
    for c in _COS_COEF[1:]:
        w = w * u + jnp.float32(c)
    w = w * _tile(alpha_ref[...])
    ws = jnp.sum(w.reshape(_B, _NSTACKS, _NPCAS), axis=1)
    e = lax.dot(ws, jnp.ones((_NPCAS, 1), jnp.float32),
                precision=jax.lax.Precision.HIGHEST)
    e_ref[...] = e.reshape(1, 1, _B)


_NSUB = 16
_CHUNK = _N_ATOMS // _NSUB
_L = 16


def _sc_segsum(e_hbm, mol_hbm, out_hbm, e_v, mol_v, acc2_v, part_v, stage_v,
               shared):
    c = lax.axis_index("c")
    s = lax.axis_index("s")

    @pl.when(c == 0)
    def _():
        base = s * _CHUNK
        pltpu.sync_copy(e_hbm.at[pl.ds(base, _CHUNK)], e_v)
        pltpu.sync_copy(mol_hbm.at[pl.ds(base, _CHUNK)], mol_v)

        zero16 = jnp.zeros((_L,), jnp.float32)
        for j in range(_L * _N_MOLS // _L):
            acc2_v[pl.ds(j * _L, _L)] = zero16

        rowoff = lax.iota(jnp.int32, _L) * _N_MOLS

        def body(i, carry):
            ids = mol_v[pl.ds(i * _L, _L)]
            vals = e_v[pl.ds(i * _L, _L)]
            plsc.addupdate_scatter(acc2_v, [ids + rowoff], vals)
            return carry

        lax.fori_loop(0, _CHUNK // _L, body, 0)

        for k in range(_N_MOLS // _L):
            ssum = zero16
            for r in range(_L):
                ssum = ssum + acc2_v[pl.ds(r * _N_MOLS + k * _L, _L)]
            part_v[pl.ds(k * _L, _L)] = ssum

        pltpu.sync_copy(part_v, shared.at[s])
        plsc.subcore_barrier()

        @pl.when(s == 0)
        def _():
            pltpu.sync_copy(shared, stage_v)
            for k in range(_N_MOLS // _L):
                ssum2 = jnp.zeros((_L,), jnp.float32)
                for r in range(_NSUB):
                    ssum2 = ssum2 + stage_v[r, pl.ds(k * _L, _L)]
                part_v[pl.ds(k * _L, _L)] = ssum2
            pltpu.sync_copy(part_v, out_hbm)


_SC_SEGSUM_CACHE = []


def _get_sc_segsum():
    if not _SC_SEGSUM_CACHE:
        k = functools.partial(
            pl.kernel,
            mesh=plsc.VectorSubcoreMesh(core_axis_name="c",
                                        subcore_axis_name="s"),
            out_type=jax.ShapeDtypeStruct((_N_MOLS,), jnp.float32),
            scratch_types=[
                pltpu.VMEM((_CHUNK,), jnp.float32),
                pltpu.VMEM((_CHUNK,), jnp.int32),
                pltpu.VMEM((_L * _N_MOLS,), jnp.float32),
                pltpu.VMEM((_N_MOLS,), jnp.float32),
                pltpu.VMEM((_NSUB, _N_MOLS), jnp.float32),
                pltpu.VMEM_SHARED((_NSUB, _N_MOLS), jnp.float32),
            ],
            compiler_params=pltpu.CompilerParams(needs_layout_passes=False),
        )(_sc_segsum)
        _SC_SEGSUM_CACHE.append(k)
    return _SC_SEGSUM_CACHE[0]


def kernel(rep, Dmat, bias, alpha, Z, mol_ids):
    hn = jnp.asarray(_hadamard(_NPCAS) / np.sqrt(_NPCAS),
                     dtype=jnp.float32).astype(jnp.bfloat16)
    alpha_s = (alpha * np.float32(np.sqrt(2.0 / _NFEAT))).reshape(
        _NSTACKS, _NPCAS)

    d0 = Dmat[:, 0].astype(jnp.bfloat16)
    d1 = Dmat[:, 1].astype(jnp.bfloat16)
    bias2_t = (bias * np.float32(1.0 / (2.0 * np.pi))).reshape(
        _N_ELEM, _NSTACKS, _NPCAS)
    z3 = Z.reshape(_NBLK, 1, _B)

    e = pl.pallas_call(
        _tc_body,
        grid=(_NBLK,),
        in_specs=[
            pl.BlockSpec((_B, _NPCAS), lambda i: (i, 0)),
            pl.BlockSpec((_N_ELEM, _NSTACKS, _NPCAS), lambda i: (0, 0, 0)),
            pl.BlockSpec((_N_ELEM, _NSTACKS, _NPCAS), lambda i: (0, 0, 0)),
            pl.BlockSpec((_N_ELEM, _NSTACKS, _NPCAS), lambda i: (0, 0, 0)),
            pl.BlockSpec((_NSTACKS, _NPCAS), lambda i: (0, 0)),
            pl.BlockSpec((_NPCAS, _NPCAS), lambda i: (0, 0)),
            pl.BlockSpec((1, 1, _B), lambda i: (i, 0, 0)),
        ],
        out_specs=pl.BlockSpec((1, 1, _B), lambda i: (i, 0, 0)),
        out_shape=jax.ShapeDtypeStruct((_NBLK, 1, _B), jnp.float32),
        compiler_params=pltpu.CompilerParams(
            dimension_semantics=("parallel",),
        ),
    )(rep, d0, d1, bias2_t, alpha_s, hn, z3)

    return _get_sc_segsum()(e.reshape(_N_ATOMS), mol_ids)
